# baseline (speedup 1.0000x reference)
import numpy as np
import jax
import jax.numpy as jnp
from jax.experimental import pallas as pl
from jax.experimental.pallas import tpu as pltpu

_T_VALUES = (2.0, 8.0, 32.0, 128.0)
_EPS = 1e-05
_W = 256


def _s_vals():
    t = np.asarray(_T_VALUES, dtype=np.float64)
    return (np.sqrt(1.0 + 4.0 * t * t) - 1.0) / (2.0 * t * t)


_S = _s_vals()


def _aug_matrix():
    i = np.arange(_W)[:, None]
    j = np.arange(_W)[None, :]
    d = np.maximum(j - i, 0)
    u = np.zeros((2 * _W, 4 * _W), dtype=np.float64)
    for t, s in enumerate(_S):
        sl = slice(t * _W, (t + 1) * _W)
        u[:_W, sl] = np.where(j >= i, s * np.exp(np.log1p(-s) * d), 0.0)
        dvec = np.exp(np.log1p(-s) * (np.arange(_W) + 1.0))
        u[_W + t, sl] = dvec
        u[_W + 4, sl] = _EPS * (1.0 - dvec)
    return u.astype(np.float32)


_U_AUG = _aug_matrix()


def _mrpcen_body(xmm_ref, xpw_ref, u_ref, p_ref, o_ref, ma_ref, mb_ref, carry_ref):
    k = pl.program_id(0)
    gdim, fdim, w = xmm_ref.shape
    rows = gdim * fdim
    xmm = xmm_ref[...].reshape(rows, w)

    @pl.when(k == 0)
    def _():
        lane = jax.lax.broadcasted_iota(jnp.int32, (rows, _W), 1)
        carry_ref[...] = jnp.where(
            lane < 4,
            xmm[:, 0:1] + _EPS,
            jnp.where(lane == 4, 1.0, 0.0),
        )

    def do_matmul(m_w_ref):
        lhs = jnp.concatenate([xmm, carry_ref[...]], axis=1)
        me_all = jax.lax.dot_general(
            lhs,
            u_ref[...],
            (((1,), (0,)), ((), ())),
            preferred_element_type=jnp.float32,
        )
        m_w_ref[...] = me_all
        new_c = [me_all[:, (t + 1) * _W - 1 : (t + 1) * _W] for t in range(4)]
        carry_ref[:, 0:4] = jnp.concatenate(new_c, axis=1)

    def do_pointwise(m_r_ref):
        xpw = xpw_ref[...].reshape(rows, w)
        nalpha2 = p_ref[0]
        delta = p_ref[1]
        r2 = p_ref[2]
        dr = p_ref[3]
        for t in range(4):
            me = m_r_ref[:, t * _W : (t + 1) * _W]
            smooth = jnp.exp2(nalpha2 * jnp.log(me))
            pcen = jnp.exp2(r2 * jnp.log(xpw * smooth + delta)) - dr
            o_ref[:, t] = pcen.reshape(gdim, fdim, w)

    @pl.when(k % 2 == 0)
    def _():
        do_matmul(ma_ref)

        @pl.when(k > 0)
        def _():
            do_pointwise(mb_ref)

    @pl.when(k % 2 == 1)
    def _():
        do_matmul(mb_ref)
        do_pointwise(ma_ref)


def kernel(x, log_alpha, log_delta, log_r):
    B, F, N = x.shape
    rows = B * F
    nk = N // _W
    alpha = jnp.exp(log_alpha)
    delta = jnp.exp(log_delta)
    r = jnp.exp(log_r)
    dr = delta**r
    inv_ln2 = float(1.0 / np.log(2.0))
    params = jnp.stack([-alpha * inv_ln2, delta, r * inv_ln2, dr])
    params = jnp.broadcast_to(params[:, None, :, None], (4, B, F, _W))
    params = params.reshape(4, rows, _W)
    u = jnp.asarray(_U_AUG)

    return pl.pallas_call(
        _mrpcen_body,
        grid=(nk + 1,),
        in_specs=[
            pl.BlockSpec((B, F, _W), lambda k: (0, 0, jnp.minimum(k, nk - 1))),
            pl.BlockSpec((B, F, _W), lambda k: (0, 0, jnp.maximum(k - 1, 0))),
            pl.BlockSpec((2 * _W, 4 * _W), lambda k: (0, 0)),
            pl.BlockSpec((4, rows, _W), lambda k: (0, 0, 0)),
        ],
        out_specs=pl.BlockSpec(
            (B, 4, F, _W), lambda k: (0, 0, 0, jnp.maximum(k - 1, 0))
        ),
        out_shape=jax.ShapeDtypeStruct((B, 4, F, N), x.dtype),
        scratch_shapes=[
            pltpu.VMEM((rows, 4 * _W), jnp.float32),
            pltpu.VMEM((rows, 4 * _W), jnp.float32),
            pltpu.VMEM((rows, _W), jnp.float32),
        ],
        compiler_params=pltpu.CompilerParams(
            dimension_semantics=("arbitrary",),
            vmem_limit_bytes=48 * 1024 * 1024,
        ),
    )(x, x, u, params)


# speedup vs baseline: 1.3270x; 1.3270x over previous
import numpy as np
import jax
import jax.numpy as jnp
from jax.experimental import pallas as pl
from jax.experimental.pallas import tpu as pltpu

_T_VALUES = (2.0, 8.0, 32.0, 128.0)
_EPS = 1e-05
_W = 256


def _s_vals():
    t = np.asarray(_T_VALUES, dtype=np.float64)
    return (np.sqrt(1.0 + 4.0 * t * t) - 1.0) / (2.0 * t * t)


_S = _s_vals()


def _aug_matrix():
    i = np.arange(_W)[:, None]
    j = np.arange(_W)[None, :]
    d = np.maximum(j - i, 0)
    u = np.zeros((2 * _W, 4 * _W), dtype=np.float64)
    for t, s in enumerate(_S):
        sl = slice(t * _W, (t + 1) * _W)
        u[:_W, sl] = np.where(j >= i, s * np.exp(np.log1p(-s) * d), 0.0)
        dvec = np.exp(np.log1p(-s) * (np.arange(_W) + 1.0))
        u[_W + t, sl] = dvec
        u[_W + 4, sl] = _EPS * (1.0 - dvec)
    return u.astype(np.float32)


_U_AUG = _aug_matrix()


def _mrpcen_body(x_ref, u_ref, p_ref, o_ref, carry_ref):
    k = pl.program_id(0)
    gdim, fdim, w = x_ref.shape
    rows = gdim * fdim
    xb = x_ref[...].reshape(rows, w)

    @pl.when(k == 0)
    def _():
        lane = jax.lax.broadcasted_iota(jnp.int32, (rows, _W), 1)
        carry_ref[...] = jnp.where(
            lane < 4,
            xb[:, 0:1] + _EPS,
            jnp.where(lane == 4, 1.0, 0.0),
        )

    lhs = jnp.concatenate([xb, carry_ref[...]], axis=1)
    nalpha2 = p_ref[0]
    delta = p_ref[1]
    r2 = p_ref[2]
    dr = p_ref[3]

    def dot_t(t):
        return jax.lax.dot_general(
            lhs,
            u_ref[:, t * _W : (t + 1) * _W],
            (((1,), (0,)), ((), ())),
            preferred_element_type=jnp.float32,
        )

    def pw_t(t, me):
        smooth = jnp.exp2(nalpha2 * jnp.log(me))
        pcen = jnp.exp2(r2 * jnp.log(xb * smooth + delta)) - dr
        o_ref[:, t] = pcen.reshape(gdim, fdim, w)

    me = [None] * 4
    me[0] = dot_t(0)
    me[1] = dot_t(1)
    pw_t(0, me[0])
    me[2] = dot_t(2)
    pw_t(1, me[1])
    me[3] = dot_t(3)
    pw_t(2, me[2])
    pw_t(3, me[3])

    carry_ref[:, 0:4] = jnp.concatenate(
        [me[t][:, _W - 1 : _W] for t in range(4)], axis=1
    )


def kernel(x, log_alpha, log_delta, log_r):
    B, F, N = x.shape
    rows = B * F
    alpha = jnp.exp(log_alpha)
    delta = jnp.exp(log_delta)
    r = jnp.exp(log_r)
    dr = delta**r
    inv_ln2 = float(1.0 / np.log(2.0))
    params = jnp.stack([-alpha * inv_ln2, delta, r * inv_ln2, dr])
    params = jnp.broadcast_to(params[:, None, :, None], (4, B, F, _W))
    params = params.reshape(4, rows, _W)
    u = jnp.asarray(_U_AUG)

    return pl.pallas_call(
        _mrpcen_body,
        grid=(N // _W,),
        in_specs=[
            pl.BlockSpec((B, F, _W), lambda k: (0, 0, k)),
            pl.BlockSpec((2 * _W, 4 * _W), lambda k: (0, 0)),
            pl.BlockSpec((4, rows, _W), lambda k: (0, 0, 0)),
        ],
        out_specs=pl.BlockSpec((B, 4, F, _W), lambda k: (0, 0, 0, k)),
        out_shape=jax.ShapeDtypeStruct((B, 4, F, N), x.dtype),
        scratch_shapes=[pltpu.VMEM((rows, _W), jnp.float32)],
        compiler_params=pltpu.CompilerParams(
            dimension_semantics=("arbitrary",),
            vmem_limit_bytes=48 * 1024 * 1024,
        ),
    )(x, u, params)
